# baseline (device time: 28480 ns/iter reference)
import jax
import jax.numpy as jnp
from jax import lax
from jax.experimental import pallas as pl
from jax.experimental.pallas import tpu as pltpu

B, S, N = 4, 512, 1024
H, D = 8, 64
S_HALF = S // 2
ROWS = 128


def kernel(O, Wo):

    def _chunk(o_ref, wo_ref, b, start):
        part = jnp.dot(
            o_ref[b, pl.ds(start, ROWS), 0, :], wo_ref[pl.ds(0, D), :],
            preferred_element_type=jnp.float32,
        )
        for h in range(1, H):
            part += jnp.dot(
                o_ref[b, pl.ds(start, ROWS), h, :],
                wo_ref[pl.ds(h * D, D), :],
                preferred_element_type=jnp.float32,
            )
        return part

    def body(o_ref, wo_ref, out_ref, ysend, yrecv, zrecv,
             ysend_sems, yrecv_sems, zsend_sems, zrecv_sems):
        my_x = lax.axis_index("x")
        my_y = lax.axis_index("y")
        my_z = lax.axis_index("z")
        other_y = 1 - my_y
        ypartner = (my_x, other_y, my_z)
        zneighbor = (my_x, my_y, 1 - my_z)

        barrier = pltpu.get_barrier_semaphore()
        for nbr in (ypartner, zneighbor):
            pl.semaphore_signal(
                barrier, inc=1, device_id=nbr,
                device_id_type=pl.DeviceIdType.MESH,
            )
        pl.semaphore_wait(barrier, 2)

        my_start = my_y * S_HALF
        other_start = other_y * S_HALF
        zc = my_z * ROWS
        oc = (1 - my_z) * ROWS

        y_rdmas = []
        for b in range(B):
            part = _chunk(o_ref, wo_ref, b, other_start + zc)
            ysend[b] = part.astype(jnp.bfloat16)
            r = pltpu.make_async_remote_copy(
                src_ref=ysend.at[b],
                dst_ref=yrecv.at[b],
                send_sem=ysend_sems.at[b],
                recv_sem=yrecv_sems.at[b],
                device_id=ypartner,
                device_id_type=pl.DeviceIdType.MESH,
            )
            r.start()
            y_rdmas.append(r)

        z_rdmas = []
        for b in range(B):
            own = _chunk(o_ref, wo_ref, b, my_start + zc)
            y_rdmas[b].wait_recv()
            zr = pltpu.make_async_remote_copy(
                src_ref=yrecv.at[b],
                dst_ref=zrecv.at[b],
                send_sem=zsend_sems.at[b],
                recv_sem=zrecv_sems.at[b],
                device_id=zneighbor,
                device_id_type=pl.DeviceIdType.MESH,
            )
            zr.start()
            z_rdmas.append(zr)
            out_ref[b, pl.ds(zc, ROWS), :] = own + yrecv[b].astype(jnp.float32)

        for b in range(B):
            own = _chunk(o_ref, wo_ref, b, my_start + oc)
            z_rdmas[b].wait_recv()
            out_ref[b, pl.ds(oc, ROWS), :] = own + zrecv[b].astype(jnp.float32)

        for b in range(B):
            y_rdmas[b].wait_send()
            z_rdmas[b].wait_send()

    return pl.pallas_call(
        body,
        out_shape=jax.ShapeDtypeStruct((B, S_HALF, N), jnp.float32),
        in_specs=[
            pl.BlockSpec(memory_space=pltpu.VMEM),
            pl.BlockSpec(memory_space=pltpu.VMEM),
        ],
        out_specs=pl.BlockSpec(memory_space=pltpu.VMEM),
        scratch_shapes=[
            pltpu.VMEM((B, ROWS, N), jnp.bfloat16),
            pltpu.VMEM((B, ROWS, N), jnp.bfloat16),
            pltpu.VMEM((B, ROWS, N), jnp.bfloat16),
            pltpu.SemaphoreType.DMA((B,)),
            pltpu.SemaphoreType.DMA((B,)),
            pltpu.SemaphoreType.DMA((B,)),
            pltpu.SemaphoreType.DMA((B,)),
        ],
        compiler_params=pltpu.CompilerParams(collective_id=0),
    )(O, Wo)


# device time: 27465 ns/iter; 1.0370x vs baseline; 1.0370x over previous
import jax
import jax.numpy as jnp
from jax import lax
from jax.experimental import pallas as pl
from jax.experimental.pallas import tpu as pltpu

B, S, N = 4, 512, 1024
H, D = 8, 64
K = H * D
S_HALF = S // 2
ROWS = 128


def kernel(O, Wo):
    O_t = jnp.transpose(O, (0, 2, 3, 1)).reshape(B, K, S)

    def _chunk(o_ref, wo_ref, b, start):
        return lax.dot_general(
            o_ref[b, :, pl.ds(start, ROWS)], wo_ref[:, :],
            (((0,), (0,)), ((), ())),
            preferred_element_type=jnp.float32,
        )

    def body(o_ref, wo_ref, out_ref, ysend, yrecv, zrecv, vstage,
             ysend_sems, yrecv_sems, zsend_sems, zrecv_sems, out_sems):
        my_x = lax.axis_index("x")
        my_y = lax.axis_index("y")
        my_z = lax.axis_index("z")
        other_y = 1 - my_y
        ypartner = (my_x, other_y, my_z)
        zneighbor = (my_x, my_y, 1 - my_z)

        barrier = pltpu.get_barrier_semaphore()
        for nbr in (ypartner, zneighbor):
            pl.semaphore_signal(
                barrier, inc=1, device_id=nbr,
                device_id_type=pl.DeviceIdType.MESH,
            )
        pl.semaphore_wait(barrier, 2)

        my_start = my_y * S_HALF
        other_start = other_y * S_HALF
        zc = my_z * ROWS
        oc = (1 - my_z) * ROWS

        y_rdmas = []
        for b in range(B):
            part = _chunk(o_ref, wo_ref, b, other_start + zc)
            ysend[b] = part.astype(jnp.bfloat16)
            r = pltpu.make_async_remote_copy(
                src_ref=ysend.at[b],
                dst_ref=yrecv.at[b],
                send_sem=ysend_sems.at[b],
                recv_sem=yrecv_sems.at[b],
                device_id=ypartner,
                device_id_type=pl.DeviceIdType.MESH,
            )
            r.start()
            y_rdmas.append(r)

        z_rdmas = []
        out_copies = []
        for b in range(B):
            own = _chunk(o_ref, wo_ref, b, my_start + zc)
            y_rdmas[b].wait_recv()
            zr = pltpu.make_async_remote_copy(
                src_ref=yrecv.at[b],
                dst_ref=zrecv.at[b],
                send_sem=zsend_sems.at[b],
                recv_sem=zrecv_sems.at[b],
                device_id=zneighbor,
                device_id_type=pl.DeviceIdType.MESH,
            )
            zr.start()
            z_rdmas.append(zr)
            vstage[0, b] = own + yrecv[b].astype(jnp.float32)
            oc_copy = pltpu.make_async_copy(
                vstage.at[0, b],
                out_ref.at[b, pl.ds(zc, ROWS), :],
                out_sems.at[0, b],
            )
            oc_copy.start()
            out_copies.append(oc_copy)

        for b in range(B):
            own = _chunk(o_ref, wo_ref, b, my_start + oc)
            z_rdmas[b].wait_recv()
            vstage[1, b] = own + zrecv[b].astype(jnp.float32)
            oc_copy = pltpu.make_async_copy(
                vstage.at[1, b],
                out_ref.at[b, pl.ds(oc, ROWS), :],
                out_sems.at[1, b],
            )
            oc_copy.start()
            out_copies.append(oc_copy)

        for c in out_copies:
            c.wait()
        for b in range(B):
            y_rdmas[b].wait_send()
            z_rdmas[b].wait_send()

    return pl.pallas_call(
        body,
        out_shape=jax.ShapeDtypeStruct((B, S_HALF, N), jnp.float32),
        in_specs=[
            pl.BlockSpec(memory_space=pltpu.VMEM),
            pl.BlockSpec(memory_space=pltpu.VMEM),
        ],
        out_specs=pl.BlockSpec(memory_space=pl.ANY),
        scratch_shapes=[
            pltpu.VMEM((B, ROWS, N), jnp.bfloat16),
            pltpu.VMEM((B, ROWS, N), jnp.bfloat16),
            pltpu.VMEM((B, ROWS, N), jnp.bfloat16),
            pltpu.VMEM((2, B, ROWS, N), jnp.float32),
            pltpu.SemaphoreType.DMA((B,)),
            pltpu.SemaphoreType.DMA((B,)),
            pltpu.SemaphoreType.DMA((B,)),
            pltpu.SemaphoreType.DMA((B,)),
            pltpu.SemaphoreType.DMA((2, B)),
        ],
        compiler_params=pltpu.CompilerParams(collective_id=0),
    )(O_t, Wo)


# device time: 26204 ns/iter; 1.0869x vs baseline; 1.0481x over previous
import jax
import jax.numpy as jnp
from jax import lax
from jax.experimental import pallas as pl
from jax.experimental.pallas import tpu as pltpu

B, S, N = 4, 512, 1024
H, D = 8, 64
K = H * D
S_HALF = S // 2
ROWS = 128
SUB = 64
NSUB = ROWS // SUB


def kernel(O, Wo):
    O_t = jnp.transpose(O, (0, 2, 3, 1)).reshape(B, K, S)

    def _chunk(o_ref, wo_ref, b, start):
        return lax.dot_general(
            o_ref[b, :, pl.ds(start, ROWS)], wo_ref[:, :],
            (((0,), (0,)), ((), ())),
            preferred_element_type=jnp.float32,
        )

    def body(o_ref, wo_ref, out_ref, ysend, yrecv, zrecv, vstage,
             ysend_sems, yrecv_sems, zsend_sems, zrecv_sems, out_sems):
        my_x = lax.axis_index("x")
        my_y = lax.axis_index("y")
        my_z = lax.axis_index("z")
        other_y = 1 - my_y
        ypartner = (my_x, other_y, my_z)
        zneighbor = (my_x, my_y, 1 - my_z)

        barrier = pltpu.get_barrier_semaphore()
        for nbr in (ypartner, zneighbor):
            pl.semaphore_signal(
                barrier, inc=1, device_id=nbr,
                device_id_type=pl.DeviceIdType.MESH,
            )
        pl.semaphore_wait(barrier, 2)

        my_start = my_y * S_HALF
        other_start = other_y * S_HALF
        zc = my_z * ROWS
        oc = (1 - my_z) * ROWS

        y_rdmas = {}
        for b in range(B):
            part = _chunk(o_ref, wo_ref, b, other_start + zc)
            ysend[b] = part.astype(jnp.bfloat16)
            for j in range(NSUB):
                r = pltpu.make_async_remote_copy(
                    src_ref=ysend.at[b, pl.ds(j * SUB, SUB), :],
                    dst_ref=yrecv.at[b, pl.ds(j * SUB, SUB), :],
                    send_sem=ysend_sems.at[b, j],
                    recv_sem=yrecv_sems.at[b, j],
                    device_id=ypartner,
                    device_id_type=pl.DeviceIdType.MESH,
                )
                r.start()
                y_rdmas[b, j] = r

        z_rdmas = {}
        out_copies = []
        for b in range(B):
            own = _chunk(o_ref, wo_ref, b, my_start + zc)
            for j in range(NSUB):
                y_rdmas[b, j].wait_recv()
                zr = pltpu.make_async_remote_copy(
                    src_ref=yrecv.at[b, pl.ds(j * SUB, SUB), :],
                    dst_ref=zrecv.at[b, pl.ds(j * SUB, SUB), :],
                    send_sem=zsend_sems.at[b, j],
                    recv_sem=zrecv_sems.at[b, j],
                    device_id=zneighbor,
                    device_id_type=pl.DeviceIdType.MESH,
                )
                zr.start()
                z_rdmas[b, j] = zr
            vstage[0, b] = own + yrecv[b].astype(jnp.float32)
            oc_copy = pltpu.make_async_copy(
                vstage.at[0, b],
                out_ref.at[b, pl.ds(zc, ROWS), :],
                out_sems.at[0, b],
            )
            oc_copy.start()
            out_copies.append(oc_copy)

        for b in range(B):
            own = _chunk(o_ref, wo_ref, b, my_start + oc)
            for j in range(NSUB):
                z_rdmas[b, j].wait_recv()
            vstage[1, b] = own + zrecv[b].astype(jnp.float32)
            oc_copy = pltpu.make_async_copy(
                vstage.at[1, b],
                out_ref.at[b, pl.ds(oc, ROWS), :],
                out_sems.at[1, b],
            )
            oc_copy.start()
            out_copies.append(oc_copy)

        for c in out_copies:
            c.wait()
        for b in range(B):
            for j in range(NSUB):
                y_rdmas[b, j].wait_send()
                z_rdmas[b, j].wait_send()

    return pl.pallas_call(
        body,
        out_shape=jax.ShapeDtypeStruct((B, S_HALF, N), jnp.float32),
        in_specs=[
            pl.BlockSpec(memory_space=pltpu.VMEM),
            pl.BlockSpec(memory_space=pltpu.VMEM),
        ],
        out_specs=pl.BlockSpec(memory_space=pltpu.MemorySpace.HBM),
        scratch_shapes=[
            pltpu.VMEM((B, ROWS, N), jnp.bfloat16),
            pltpu.VMEM((B, ROWS, N), jnp.bfloat16),
            pltpu.VMEM((B, ROWS, N), jnp.bfloat16),
            pltpu.VMEM((2, B, ROWS, N), jnp.float32),
            pltpu.SemaphoreType.DMA((B, NSUB)),
            pltpu.SemaphoreType.DMA((B, NSUB)),
            pltpu.SemaphoreType.DMA((B, NSUB)),
            pltpu.SemaphoreType.DMA((B, NSUB)),
            pltpu.SemaphoreType.DMA((2, B)),
        ],
        compiler_params=pltpu.CompilerParams(collective_id=0),
    )(O_t, Wo)
